# Initial kernel scaffold; baseline (speedup 1.0000x reference)
import jax
import jax.numpy as jnp
from jax.experimental import pallas as pl


def kernel(x, edge_index, W1, b1, W2, b2, W3, b3, W_ih, W_hh, b_ih, b_hh, Wp1, bp1, Wp2, bp2, Wp3, bp3):
    raise NotImplementedError("write your pallas kernel here")


# speedup vs baseline: 28.0107x; 28.0107x over previous
import functools

import jax
import jax.numpy as jnp
from jax import lax
from jax.experimental import pallas as pl
from jax.experimental.pallas import tpu as pltpu
from jax.experimental.pallas import tpu_sc as plsc

B, T, N, F, D, OUT = 2, 4, 10000, 128, 64, 8
E = 160000
G = B * T
P = G // 2
DP = 2 * D
N_PAD = 10240
E_PAD = 163840
MAIN_CHUNKS = 80
DEG_CHUNKS = 40
RPT = N_PAD // 16
P_PER_CORE = P // 2
BN = 1024


def _mesh():
    return plsc.VectorSubcoreMesh(core_axis_name="c", subcore_axis_name="s")


@functools.partial(
    pl.kernel,
    out_type=jax.ShapeDtypeStruct((2, N_PAD, 16), jnp.float32),
    mesh=_mesh(),
    scratch_types=[
        pltpu.VMEM((DEG_CHUNKS, 128), jnp.int32),
        pltpu.VMEM((128, 16), jnp.float32),
        pltpu.VMEM_SHARED((N_PAD, 16), jnp.float32),
    ],
)
def _sc_degree(dst_hbm, zeros_hbm, ones_hbm, out_hbm, dst_v, ones_v, acc):
    c = lax.axis_index("c")
    s = lax.axis_index("s")
    tid = c * 16 + s
    pltpu.sync_copy(dst_hbm.at[tid], dst_v)
    pltpu.sync_copy(ones_hbm, ones_v)
    pltpu.sync_copy(zeros_hbm.at[pl.ds(s * RPT, RPT)], acc.at[pl.ds(s * RPT, RPT)])
    plsc.subcore_barrier()

    def body(j, carry):
        pltpu.sync_copy(ones_v, acc.at[dst_v.at[j]], add=True)
        return carry

    lax.fori_loop(0, DEG_CHUNKS, body, 0)
    plsc.subcore_barrier()
    pltpu.sync_copy(acc.at[pl.ds(s * RPT, RPT)], out_hbm.at[c].at[pl.ds(s * RPT, RPT)])


@functools.partial(
    pl.kernel,
    out_type=jax.ShapeDtypeStruct((P, N_PAD, DP), jnp.float32),
    mesh=_mesh(),
    scratch_types=[
        pltpu.VMEM((MAIN_CHUNKS, 128), jnp.int32),
        pltpu.VMEM((MAIN_CHUNKS, 128), jnp.int32),
        pltpu.VMEM((128, DP), jnp.float32),
        pltpu.VMEM_SHARED((N_PAD, DP), jnp.float32),
        pltpu.SemaphoreType.DMA,
    ],
)
def _sc_aggregate(mt_hbm, src_hbm, dst_hbm, out_hbm, src_v, dst_v, rows_v, acc, sem):
    c = lax.axis_index("c")
    s = lax.axis_index("s")
    pltpu.sync_copy(src_hbm.at[s], src_v)
    pltpu.sync_copy(dst_hbm.at[s], dst_v)
    for pi in range(P_PER_CORE):
        p = c * P_PER_CORE + pi
        pltpu.sync_copy(mt_hbm.at[p].at[pl.ds(s * RPT, RPT)], acc.at[pl.ds(s * RPT, RPT)])
        plsc.subcore_barrier()

        def body(j, carry):
            pltpu.async_copy(mt_hbm.at[p].at[src_v.at[j]], rows_v, sem).wait()
            pltpu.sync_copy(rows_v, acc.at[dst_v.at[j]], add=True)
            return carry

        lax.fori_loop(0, MAIN_CHUNKS, body, 0)
        plsc.subcore_barrier()
        pltpu.sync_copy(acc.at[pl.ds(s * RPT, RPT)], out_hbm.at[p].at[pl.ds(s * RPT, RPT)])


def _dis_body(cnt_ref, o_ref):
    c = cnt_ref[0, :, 0:1] + cnt_ref[1, :, 0:1]
    o_ref[...] = lax.rsqrt(1.0 + c)


def _mm1_body(xa_ref, xb_ref, w_ref, dis_ref, o_ref):
    mma = jnp.dot(xa_ref[0], w_ref[...], preferred_element_type=jnp.float32)
    mmb = jnp.dot(xb_ref[0], w_ref[...], preferred_element_type=jnp.float32)
    o_ref[0] = dis_ref[...] * jnp.concatenate([mma, mmb], axis=1)


def _mid_body(agg_ref, dis_ref, b_ref, w_ref, o_ref):
    dis = dis_ref[...]
    z = dis * agg_ref[0] + b_ref[...]
    z = z * jax.nn.sigmoid(z)
    o_ref[0] = dis * jnp.dot(z, w_ref[...], preferred_element_type=jnp.float32)


def _pool_body(agg_ref, dis_ref, b_ref, o_ref):
    i = pl.program_id(1)
    z = dis_ref[...] * agg_ref[0] + b_ref[...]
    z = z * jax.nn.sigmoid(z)
    row = i * BN + lax.broadcasted_iota(jnp.int32, (BN, 1), 0)
    z = jnp.where(row < N, z, 0.0)
    part = (jnp.sum(z, axis=0, keepdims=True) * (1.0 / N)).reshape(1, 1, DP)

    @pl.when(i == 0)
    def _():
        o_ref[...] = part

    @pl.when(i > 0)
    def _():
        o_ref[...] += part


def _lstm_head_body(p_ref, wih_ref, whh_ref, bias_ref,
                    wp1_ref, bp1_ref, wp2_ref, bp2_ref, wp3_ref, bp3_ref, o_ref):
    h = jnp.zeros((B, D), jnp.float32)
    c = jnp.zeros((B, D), jnp.float32)
    for t in range(T):
        xt = p_ref[t]
        gs = []
        for k in range(4):
            gk = (jnp.dot(xt, wih_ref[k], preferred_element_type=jnp.float32)
                  + jnp.dot(h, whh_ref[k], preferred_element_type=jnp.float32)
                  + bias_ref[k])
            gs.append(gk)
        i_g, f_g, g_g, o_g = gs
        c = jax.nn.sigmoid(f_g) * c + jax.nn.sigmoid(i_g) * jnp.tanh(g_g)
        h = jax.nn.sigmoid(o_g) * jnp.tanh(c)
    z = jnp.dot(h, wp1_ref[...], preferred_element_type=jnp.float32) + bp1_ref[...]
    z = z * jax.nn.sigmoid(z)
    z = jnp.dot(z, wp2_ref[...], preferred_element_type=jnp.float32) + bp2_ref[...]
    z = z * jax.nn.sigmoid(z)
    o_ref[...] = jnp.dot(z, wp3_ref[...], preferred_element_type=jnp.float32) + bp3_ref[...]


def _blockdiag(W):
    Z = jnp.zeros((D, D), W.dtype)
    return jnp.concatenate(
        [jnp.concatenate([W, Z], axis=1), jnp.concatenate([Z, W], axis=1)], axis=0)


def kernel(x, edge_index, W1, b1, W2, b2, W3, b3, W_ih, W_hh, b_ih, b_hh,
           Wp1, bp1, Wp2, bp2, Wp3, bp3):
    nb = N_PAD // BN

    xg = x.reshape(G, N, F)
    x_pad = jnp.pad(xg, ((0, 0), (0, N_PAD - N), (0, 0)))
    src = edge_index[0]
    dst = edge_index[1]
    src_pad = jnp.concatenate([src, jnp.zeros((E_PAD - E,), src.dtype)])
    dst_pad = jnp.concatenate([dst, jnp.full((E_PAD - E,), N, dst.dtype)])
    src_m = src_pad.reshape(16, MAIN_CHUNKS, 128)
    dst_m = dst_pad.reshape(16, MAIN_CHUNKS, 128)
    dst_d = dst_pad.reshape(32, DEG_CHUNKS, 128)
    zeros16 = jnp.zeros((N_PAD, 16), jnp.float32)
    ones16 = jnp.ones((128, 16), jnp.float32)

    cnt = _sc_degree(dst_d, zeros16, ones16)
    dis2 = pl.pallas_call(
        _dis_body,
        grid=(nb,),
        in_specs=[pl.BlockSpec((2, BN, 16), lambda i: (0, i, 0))],
        out_specs=pl.BlockSpec((BN, 1), lambda i: (i, 0)),
        out_shape=jax.ShapeDtypeStruct((N_PAD, 1), jnp.float32),
    )(cnt)

    mt = pl.pallas_call(
        _mm1_body,
        grid=(P, nb),
        in_specs=[
            pl.BlockSpec((1, BN, F), lambda p, i: (2 * p, i, 0)),
            pl.BlockSpec((1, BN, F), lambda p, i: (2 * p + 1, i, 0)),
            pl.BlockSpec((F, D), lambda p, i: (0, 0)),
            pl.BlockSpec((BN, 1), lambda p, i: (i, 0)),
        ],
        out_specs=pl.BlockSpec((1, BN, DP), lambda p, i: (p, i, 0)),
        out_shape=jax.ShapeDtypeStruct((P, N_PAD, DP), jnp.float32),
    )(x_pad, x_pad, W1, dis2)

    def mid(agg, b, W):
        return pl.pallas_call(
            _mid_body,
            grid=(P, nb),
            in_specs=[
                pl.BlockSpec((1, BN, DP), lambda p, i: (p, i, 0)),
                pl.BlockSpec((BN, 1), lambda p, i: (i, 0)),
                pl.BlockSpec((1, DP), lambda p, i: (0, 0)),
                pl.BlockSpec((DP, DP), lambda p, i: (0, 0)),
            ],
            out_specs=pl.BlockSpec((1, BN, DP), lambda p, i: (p, i, 0)),
            out_shape=jax.ShapeDtypeStruct((P, N_PAD, DP), jnp.float32),
        )(agg, dis2, jnp.concatenate([b, b]).reshape(1, DP), _blockdiag(W))

    agg = _sc_aggregate(mt, src_m, dst_m)
    mt = mid(agg, b1, W2)
    agg = _sc_aggregate(mt, src_m, dst_m)
    mt = mid(agg, b2, W3)
    agg = _sc_aggregate(mt, src_m, dst_m)

    pooled = pl.pallas_call(
        _pool_body,
        grid=(P, nb),
        in_specs=[
            pl.BlockSpec((1, BN, DP), lambda p, i: (p, i, 0)),
            pl.BlockSpec((BN, 1), lambda p, i: (i, 0)),
            pl.BlockSpec((1, DP), lambda p, i: (0, 0)),
        ],
        out_specs=pl.BlockSpec((1, 1, DP), lambda p, i: (p, 0, 0)),
        out_shape=jax.ShapeDtypeStruct((P, 1, DP), jnp.float32),
    )(agg, dis2, jnp.concatenate([b3, b3]).reshape(1, DP))

    p_tb = pooled.reshape(B, T, D).transpose(1, 0, 2)
    wih = jnp.stack([W_ih[k * D:(k + 1) * D].T for k in range(4)])
    whh = jnp.stack([W_hh[k * D:(k + 1) * D].T for k in range(4)])
    bias = (b_ih + b_hh).reshape(4, 1, D)
    out = pl.pallas_call(
        _lstm_head_body,
        out_shape=jax.ShapeDtypeStruct((B, OUT), jnp.float32),
    )(p_tb, wih, whh, bias, Wp1, bp1.reshape(1, 2 * D),
      Wp2, bp2.reshape(1, D), Wp3, bp3.reshape(1, OUT))
    return out
